# Initial kernel scaffold; baseline (speedup 1.0000x reference)
import jax
import jax.numpy as jnp
from jax.experimental import pallas as pl


def kernel(x, W0, b0, vals1, rows1, cols1, vals2, rows2, cols2):
    raise NotImplementedError("write your pallas kernel here")


# speedup vs baseline: 1.2728x; 1.2728x over previous
import functools

import jax
import jax.numpy as jnp
from jax import lax
from jax.experimental import pallas as pl
from jax.experimental.pallas import tpu as pltpu
from jax.experimental.pallas import tpu_sc as plsc

BATCH = 1024
NC = 2
NS = 16
NW = NC * NS


def _dense_body(w_ref, x_ref, b_ref, out_ref):
    i = pl.program_id(0)
    acc = lax.dot_general(
        w_ref[...], x_ref[...],
        dimension_numbers=(((1,), (1,)), ((), ())),
        preferred_element_type=jnp.float32,
    )
    bias = b_ref[0, pl.ds(i * w_ref.shape[0], w_ref.shape[0])]
    out_ref[...] = jnp.maximum(acc + bias[:, None], 0.0)


def _dense_tc(x, W0, b0):
    D, K = W0.shape
    BM = 256
    grid = (D // BM,)
    return pl.pallas_call(
        _dense_body,
        grid=grid,
        in_specs=[
            pl.BlockSpec((BM, K), lambda i: (i, 0)),
            pl.BlockSpec((BATCH, K), lambda i: (0, 0)),
            pl.BlockSpec((1, D), lambda i: (0, 0)),
        ],
        out_specs=pl.BlockSpec((BM, BATCH), lambda i: (i, 0)),
        out_shape=jax.ShapeDtypeStruct((D, BATCH), jnp.float32),
    )(W0, x, b0.reshape(1, D))


def _spmm_body(M, RB, NB, CH, RPW, relu_in,
               h_hbm, vals_hbm, rows_hbm, cols_hbm, bptr_hbm, zeros_hbm,
               out_hbm, bptr_v, rowbuf, valbuf, colbuf, gbuf, acc, sem):
    wid = lax.axis_index("s") * NC + lax.axis_index("c")
    pltpu.sync_copy(bptr_hbm, bptr_v)

    def blk_body(blk, _):
        r0 = wid * RPW + blk * RB
        bp = bptr_v[pl.ds(wid * NB + blk, 16)]
        e0 = bp[0]
        e1 = bp[1]
        e0a = (e0 // 8) * 8
        nch = (e1 - e0a + CH - 1) // CH
        pltpu.sync_copy(zeros_hbm, acc)

        def chunk_body(c, _):
            s = e0a + c * CH
            pltpu.sync_copy(rows_hbm.at[pl.ds(s, CH)], rowbuf.at[pl.ds(0, CH)])
            pltpu.sync_copy(vals_hbm.at[pl.ds(s, CH)], valbuf.at[pl.ds(0, CH)])
            pltpu.sync_copy(cols_hbm.at[pl.ds(s, CH)], colbuf)
            pltpu.async_copy(h_hbm.at[colbuf], gbuf, sem).wait()

            def ent_body(e, _):
                lr = rowbuf[pl.ds(e, 16)][0] - r0
                ok = jnp.logical_and(lr >= 0, lr < RB)
                v = jnp.where(ok, valbuf[pl.ds(e, 16)][0], 0.0)
                lrc = jnp.where(ok, lr, 0)
                vvec = jnp.full((16,), v, jnp.float32)
                base = lrc * BATCH
                for j in range(BATCH // 16):
                    g = gbuf[e, pl.ds(j * 16, 16)]
                    if relu_in:
                        g = jnp.maximum(g, 0.0)
                    plsc.addupdate(acc.at[pl.ds(base + j * 16, 16)], vvec * g)
                return 0

            lax.fori_loop(0, CH, ent_body, 0)
            return 0

        lax.fori_loop(0, nch, chunk_body, 0)
        pltpu.sync_copy(acc, out_hbm.at[pl.ds(r0 * BATCH, RB * BATCH)])
        return 0

    lax.fori_loop(0, NB, blk_body, 0)


def _spmm_sc(h, vals, rows, cols, M, RB, CH, relu_in):
    nnz = vals.shape[0]
    RPW = M // NW
    NB = RPW // RB
    bptr = jnp.searchsorted(rows, jnp.arange(0, M + 1, RB)).astype(jnp.int32)
    PB = ((M // RB + 1 + 16 + 7) // 8) * 8
    bptr = jnp.concatenate(
        [bptr, jnp.full((PB - bptr.shape[0],), nnz, jnp.int32)])
    L = (((nnz + CH) + 7) // 8) * 8
    pad = L - nnz
    rows_p = jnp.concatenate([rows, jnp.full((pad,), M, jnp.int32)])
    cols_p = jnp.concatenate([cols, jnp.zeros((pad,), jnp.int32)])
    vals_p = jnp.concatenate([vals, jnp.zeros((pad,), jnp.float32)])
    zeros = jnp.zeros((RB * BATCH,), jnp.float32)

    mesh = plsc.VectorSubcoreMesh(core_axis_name="c", subcore_axis_name="s",
                                  num_cores=NC, num_subcores=NS)
    body = functools.partial(_spmm_body, M, RB, NB, CH, RPW, relu_in)
    f = pl.kernel(
        body,
        out_type=jax.ShapeDtypeStruct((M * BATCH,), jnp.float32),
        mesh=mesh,
        scratch_types=[
            pltpu.VMEM((PB,), jnp.int32),
            pltpu.VMEM((CH + 16,), jnp.int32),
            pltpu.VMEM((CH + 16,), jnp.float32),
            pltpu.VMEM((CH,), jnp.int32),
            pltpu.VMEM((CH, BATCH), jnp.float32),
            pltpu.VMEM((RB * BATCH,), jnp.float32),
            pltpu.SemaphoreType.DMA,
        ],
    )
    out = f(h, vals_p, rows_p, cols_p, bptr, zeros)
    return out.reshape(M, BATCH)


def _transpose_body(in_ref, out_ref):
    out_ref[...] = jnp.maximum(in_ref[...].T, 0.0)


def _transpose_tc(h):
    D, B = h.shape
    BM, BN = 512, 256
    return pl.pallas_call(
        _transpose_body,
        grid=(D // BM, B // BN),
        in_specs=[pl.BlockSpec((BM, BN), lambda i, j: (i, j))],
        out_specs=pl.BlockSpec((BN, BM), lambda i, j: (j, i)),
        out_shape=jax.ShapeDtypeStruct((B, D), jnp.float32),
    )(h)


def kernel(x, W0, b0, vals1, rows1, cols1, vals2, rows2, cols2):
    S1 = 16384
    S2 = 4096
    h1 = _dense_tc(x, W0, b0)
    h2 = _spmm_sc(h1, vals1, rows1, cols1, M=S1, RB=32, CH=64, relu_in=False)
    h3 = _spmm_sc(h2, vals2, rows2, cols2, M=S2, RB=32, CH=64, relu_in=True)
    return _transpose_tc(h3)


# speedup vs baseline: 1.4599x; 1.1470x over previous
import functools

import jax
import jax.numpy as jnp
from jax import lax
from jax.experimental import pallas as pl
from jax.experimental.pallas import tpu as pltpu
from jax.experimental.pallas import tpu_sc as plsc

BATCH = 1024
NC = 2
NS = 16
NW = NC * NS


def _dense_body(w_ref, x_ref, b_ref, out_ref):
    i = pl.program_id(0)
    acc = lax.dot_general(
        w_ref[...], x_ref[...],
        dimension_numbers=(((1,), (1,)), ((), ())),
        preferred_element_type=jnp.float32,
    )
    bias = b_ref[0, pl.ds(i * w_ref.shape[0], w_ref.shape[0])]
    out_ref[...] = jnp.maximum(acc + bias[:, None], 0.0)


def _dense_tc(x, W0, b0):
    D, K = W0.shape
    BM = 256
    grid = (D // BM,)
    return pl.pallas_call(
        _dense_body,
        grid=grid,
        in_specs=[
            pl.BlockSpec((BM, K), lambda i: (i, 0)),
            pl.BlockSpec((BATCH, K), lambda i: (0, 0)),
            pl.BlockSpec((1, D), lambda i: (0, 0)),
        ],
        out_specs=pl.BlockSpec((BM, BATCH), lambda i: (i, 0)),
        out_shape=jax.ShapeDtypeStruct((D, BATCH), jnp.float32),
    )(W0, x, b0.reshape(1, D))


def _spmm_body(M, RB, NB, CH, RPW, relu_in,
               h_hbm, rv_hbm, cols_hbm, bptr_hbm, out_hbm,
               bptr_v, cbb0, cbb1, rvb0, rvb1, gb0, gb1, acc,
               sg0, sg1, scb0, scb1, srv0, srv1):
    wid = lax.axis_index("s") * NC + lax.axis_index("c")
    pltpu.sync_copy(bptr_hbm, bptr_v)
    zvec = jnp.zeros((16,), jnp.float32)

    def blk_body(blk, _):
        r0 = wid * RPW + blk * RB
        bp = bptr_v[pl.ds(wid * NB + blk, 16)]
        e0 = bp[0]
        e1 = bp[1]
        e0a = (e0 // 8) * 8
        nch = jnp.maximum((e1 - e0a + CH - 1) // CH, 1)
        nchp = (nch + 1) // 2

        def st(c):
            return e0a + jnp.minimum(c, nch - 1) * CH

        def cb_copy(c, cbb, sem):
            return pltpu.make_async_copy(
                cols_hbm.at[pl.ds(st(c), CH)], cbb, sem)

        def rv_copy(c, rvb, sem):
            return pltpu.make_async_copy(
                rv_hbm.at[pl.ds(2 * st(c), 2 * CH)],
                rvb.at[pl.ds(0, 2 * CH)], sem)

        def g_copy(cbb, gb, sem):
            return pltpu.make_async_copy(h_hbm.at[cbb], gb, sem)

        cb_copy(0, cbb0, scb0).start()
        rv_copy(0, rvb0, srv0).start()
        cb_copy(1, cbb1, scb1).start()
        rv_copy(1, rvb1, srv1).start()

        def zrow(i, _):
            for j in range(BATCH // 16):
                acc[pl.ds(i * BATCH + j * 16, 16)] = zvec
            return 0

        lax.fori_loop(0, RB, zrow, 0)
        cb_copy(0, cbb0, scb0).wait()
        g_copy(cbb0, gb0, sg0).start()

        def compute(gb, rvb, guard):
            def ent(e, _):
                w = rvb[pl.ds(2 * e, 16)]
                lr = w[0] - r0
                ok = jnp.logical_and(
                    jnp.logical_and(lr >= 0, lr < RB), guard)
                v = jnp.where(ok, lax.bitcast_convert_type(w[1], jnp.float32),
                              0.0)
                lrc = jnp.where(ok, lr, 0)
                vvec = jnp.full((16,), v, jnp.float32)
                base = lrc * BATCH
                for j in range(BATCH // 16):
                    g = gb[e, pl.ds(j * 16, 16)]
                    if relu_in:
                        g = jnp.maximum(g, 0.0)
                    plsc.addupdate(acc.at[pl.ds(base + j * 16, 16)], vvec * g)
                return 0

            lax.fori_loop(0, CH, ent, 0)

        def pair(t, _):
            a = 2 * t
            b = a + 1
            g_copy(cbb0, gb0, sg0).wait()
            cb_copy(b, cbb1, scb1).wait()
            g_copy(cbb1, gb1, sg1).start()
            cb_copy(a + 2, cbb0, scb0).start()
            rv_copy(a, rvb0, srv0).wait()
            compute(gb0, rvb0, True)
            rv_copy(a + 2, rvb0, srv0).start()
            g_copy(cbb1, gb1, sg1).wait()
            cb_copy(a + 2, cbb0, scb0).wait()
            g_copy(cbb0, gb0, sg0).start()
            cb_copy(b + 2, cbb1, scb1).start()
            rv_copy(b, rvb1, srv1).wait()
            compute(gb1, rvb1, b < nch)
            rv_copy(b + 2, rvb1, srv1).start()
            return 0

        lax.fori_loop(0, nchp, pair, 0)
        g_copy(cbb0, gb0, sg0).wait()
        cb_copy(0, cbb1, scb1).wait()
        rv_copy(0, rvb0, srv0).wait()
        rv_copy(0, rvb1, srv1).wait()
        pltpu.sync_copy(acc, out_hbm.at[pl.ds(r0 * BATCH, RB * BATCH)])
        return 0

    lax.fori_loop(0, NB, blk_body, 0)


def _spmm_sc(h, vals, rows, cols, M, RB, CH, relu_in):
    nnz = vals.shape[0]
    RPW = M // NW
    NB = RPW // RB
    bptr = jnp.searchsorted(rows, jnp.arange(0, M + 1, RB)).astype(jnp.int32)
    PB = ((M // RB + 1 + 16 + 7) // 8) * 8
    bptr = jnp.concatenate(
        [bptr, jnp.full((PB - bptr.shape[0],), nnz, jnp.int32)])
    L = (((nnz + 2 * CH) + 7) // 8) * 8
    pad = L - nnz
    rows_p = jnp.concatenate([rows, jnp.full((pad,), M, jnp.int32)])
    cols_p = jnp.concatenate([cols, jnp.zeros((pad,), jnp.int32)])
    vals_p = jnp.concatenate([vals, jnp.zeros((pad,), jnp.float32)])
    vbits = lax.bitcast_convert_type(vals_p, jnp.int32)
    rv = jnp.stack([rows_p, vbits], axis=1).reshape(-1)

    mesh = plsc.VectorSubcoreMesh(core_axis_name="c", subcore_axis_name="s",
                                  num_cores=NC, num_subcores=NS)
    body = functools.partial(_spmm_body, M, RB, NB, CH, RPW, relu_in)
    f = pl.kernel(
        body,
        out_type=jax.ShapeDtypeStruct((M * BATCH,), jnp.float32),
        mesh=mesh,
        scratch_types=[
            pltpu.VMEM((PB,), jnp.int32),
            pltpu.VMEM((CH,), jnp.int32),
            pltpu.VMEM((CH,), jnp.int32),
            pltpu.VMEM((2 * CH + 16,), jnp.int32),
            pltpu.VMEM((2 * CH + 16,), jnp.int32),
            pltpu.VMEM((CH, BATCH), jnp.float32),
            pltpu.VMEM((CH, BATCH), jnp.float32),
            pltpu.VMEM((RB * BATCH,), jnp.float32),
            pltpu.SemaphoreType.DMA,
            pltpu.SemaphoreType.DMA,
            pltpu.SemaphoreType.DMA,
            pltpu.SemaphoreType.DMA,
            pltpu.SemaphoreType.DMA,
            pltpu.SemaphoreType.DMA,
        ],
    )
    out = f(h, rv, cols_p, bptr)
    return out.reshape(M, BATCH)


def _transpose_body(in_ref, out_ref):
    out_ref[...] = jnp.maximum(in_ref[...].T, 0.0)


def _transpose_tc(h):
    D, B = h.shape
    BM, BN = 512, 256
    return pl.pallas_call(
        _transpose_body,
        grid=(D // BM, B // BN),
        in_specs=[pl.BlockSpec((BM, BN), lambda i, j: (i, j))],
        out_specs=pl.BlockSpec((BN, BM), lambda i, j: (j, i)),
        out_shape=jax.ShapeDtypeStruct((B, D), jnp.float32),
    )(h)


def kernel(x, W0, b0, vals1, rows1, cols1, vals2, rows2, cols2):
    S1 = 16384
    S2 = 4096
    h1 = _dense_tc(x, W0, b0)
    h2 = _spmm_sc(h1, vals1, rows1, cols1, M=S1, RB=32, CH=32, relu_in=False)
    h3 = _spmm_sc(h2, vals2, rows2, cols2, M=S2, RB=32, CH=32, relu_in=True)
    return _transpose_tc(h3)


# speedup vs baseline: 3.4652x; 2.3736x over previous
import functools

import jax
import jax.numpy as jnp
from jax import lax
from jax.experimental import pallas as pl
from jax.experimental.pallas import tpu as pltpu
from jax.experimental.pallas import tpu_sc as plsc

BATCH = 1024
NC = 2
NS = 16
NW = NC * NS


def _dense_body(w_ref, x_ref, b_ref, out_ref):
    i = pl.program_id(0)
    acc = lax.dot_general(
        w_ref[...], x_ref[...],
        dimension_numbers=(((1,), (1,)), ((), ())),
        preferred_element_type=jnp.float32,
    )
    bias = b_ref[0, pl.ds(i * w_ref.shape[0], w_ref.shape[0])]
    out_ref[...] = jnp.maximum(acc + bias[:, None], 0.0)


def _dense_tc(x, W0, b0):
    D, K = W0.shape
    BM = 256
    grid = (D // BM,)
    return pl.pallas_call(
        _dense_body,
        grid=grid,
        in_specs=[
            pl.BlockSpec((BM, K), lambda i: (i, 0)),
            pl.BlockSpec((BATCH, K), lambda i: (0, 0)),
            pl.BlockSpec((1, D), lambda i: (0, 0)),
        ],
        out_specs=pl.BlockSpec((BM, BATCH), lambda i: (i, 0)),
        out_shape=jax.ShapeDtypeStruct((D, BATCH), jnp.float32),
    )(W0, x, b0.reshape(1, D))


def _spmm_body(M, RB, NB, CH, RPW, relu_in,
               h_hbm, rv_hbm, cols_hbm, bptr_hbm, out_hbm,
               bptr_v, cbb0, cbb1, rvb0, rvb1, gb0, gb1, acc,
               sg0, sg1, scb0, scb1, srv0, srv1):
    wid = lax.axis_index("s") * NC + lax.axis_index("c")
    pltpu.sync_copy(bptr_hbm, bptr_v)
    zvec = jnp.zeros((16,), jnp.float32)

    def blk_body(blk, _):
        r0 = wid * RPW + blk * RB
        bp = bptr_v[pl.ds(wid * NB + blk, 16)]
        e0 = bp[0]
        e1 = bp[1]
        e0a = (e0 // 8) * 8
        nch = jnp.maximum((e1 - e0a + CH - 1) // CH, 1)
        nchp = (nch + 1) // 2

        def st(c):
            return e0a + jnp.minimum(c, nch - 1) * CH

        def cb_copy(c, cbb, sem):
            return pltpu.make_async_copy(
                cols_hbm.at[pl.ds(st(c), CH)], cbb, sem)

        def rv_copy(c, rvb, sem):
            return pltpu.make_async_copy(
                rv_hbm.at[pl.ds(2 * st(c), 2 * CH)],
                rvb.at[pl.ds(0, 2 * CH)], sem)

        def g_copy(cbb, gb, sem):
            return pltpu.make_async_copy(h_hbm.at[cbb], gb, sem)

        cb_copy(0, cbb0, scb0).start()
        rv_copy(0, rvb0, srv0).start()
        cb_copy(1, cbb1, scb1).start()
        rv_copy(1, rvb1, srv1).start()

        @plsc.parallel_loop(0, RB * BATCH // 16, 1, unroll=8)
        def _zero(i):
            acc[pl.ds(i * 16, 16)] = zvec
        cb_copy(0, cbb0, scb0).wait()
        g_copy(cbb0, gb0, sg0).start()

        def compute(gb, rvb, guard):
            def ent(e, _):
                w = rvb[pl.ds(2 * e, 16)]
                lr = w[0] - r0
                ok = jnp.logical_and(
                    jnp.logical_and(lr >= 0, lr < RB), guard)
                v = jnp.where(ok, lax.bitcast_convert_type(w[1], jnp.float32),
                              0.0)
                lrc = jnp.where(ok, lr, 0)
                vvec = jnp.full((16,), v, jnp.float32)
                base = lrc * BATCH

                @plsc.parallel_loop(0, BATCH // 16, 1, unroll=8)
                def _col(j):
                    g = gb[e, pl.ds(j * 16, 16)]
                    if relu_in:
                        g = jnp.maximum(g, 0.0)
                    plsc.addupdate(acc.at[pl.ds(base + j * 16, 16)], vvec * g)

                return 0

            lax.fori_loop(0, CH, ent, 0)

        def pair(t, _):
            a = 2 * t
            b = a + 1
            g_copy(cbb0, gb0, sg0).wait()
            cb_copy(b, cbb1, scb1).wait()
            g_copy(cbb1, gb1, sg1).start()
            cb_copy(a + 2, cbb0, scb0).start()
            rv_copy(a, rvb0, srv0).wait()
            compute(gb0, rvb0, True)
            rv_copy(a + 2, rvb0, srv0).start()
            g_copy(cbb1, gb1, sg1).wait()
            cb_copy(a + 2, cbb0, scb0).wait()
            g_copy(cbb0, gb0, sg0).start()
            cb_copy(b + 2, cbb1, scb1).start()
            rv_copy(b, rvb1, srv1).wait()
            compute(gb1, rvb1, b < nch)
            rv_copy(b + 2, rvb1, srv1).start()
            return 0

        lax.fori_loop(0, nchp, pair, 0)
        g_copy(cbb0, gb0, sg0).wait()
        cb_copy(0, cbb1, scb1).wait()
        rv_copy(0, rvb0, srv0).wait()
        rv_copy(0, rvb1, srv1).wait()
        pltpu.sync_copy(acc, out_hbm.at[pl.ds(r0 * BATCH, RB * BATCH)])
        return 0

    lax.fori_loop(0, NB, blk_body, 0)


def _spmm_sc(h, vals, rows, cols, M, RB, CH, relu_in):
    nnz = vals.shape[0]
    RPW = M // NW
    NB = RPW // RB
    bptr = jnp.searchsorted(rows, jnp.arange(0, M + 1, RB)).astype(jnp.int32)
    PB = ((M // RB + 1 + 16 + 7) // 8) * 8
    bptr = jnp.concatenate(
        [bptr, jnp.full((PB - bptr.shape[0],), nnz, jnp.int32)])
    L = (((nnz + 2 * CH) + 7) // 8) * 8
    pad = L - nnz
    rows_p = jnp.concatenate([rows, jnp.full((pad,), M, jnp.int32)])
    cols_p = jnp.concatenate([cols, jnp.zeros((pad,), jnp.int32)])
    vals_p = jnp.concatenate([vals, jnp.zeros((pad,), jnp.float32)])
    vbits = lax.bitcast_convert_type(vals_p, jnp.int32)
    rv = jnp.stack([rows_p, vbits], axis=1).reshape(-1)

    mesh = plsc.VectorSubcoreMesh(core_axis_name="c", subcore_axis_name="s",
                                  num_cores=NC, num_subcores=NS)
    body = functools.partial(_spmm_body, M, RB, NB, CH, RPW, relu_in)
    f = pl.kernel(
        body,
        out_type=jax.ShapeDtypeStruct((M * BATCH,), jnp.float32),
        mesh=mesh,
        scratch_types=[
            pltpu.VMEM((PB,), jnp.int32),
            pltpu.VMEM((CH,), jnp.int32),
            pltpu.VMEM((CH,), jnp.int32),
            pltpu.VMEM((2 * CH + 16,), jnp.int32),
            pltpu.VMEM((2 * CH + 16,), jnp.int32),
            pltpu.VMEM((CH, BATCH), jnp.float32),
            pltpu.VMEM((CH, BATCH), jnp.float32),
            pltpu.VMEM((RB * BATCH,), jnp.float32),
            pltpu.SemaphoreType.DMA,
            pltpu.SemaphoreType.DMA,
            pltpu.SemaphoreType.DMA,
            pltpu.SemaphoreType.DMA,
            pltpu.SemaphoreType.DMA,
            pltpu.SemaphoreType.DMA,
        ],
    )
    out = f(h, rv, cols_p, bptr)
    return out.reshape(M, BATCH)


def _transpose_body(in_ref, out_ref):
    out_ref[...] = jnp.maximum(in_ref[...].T, 0.0)


def _transpose_tc(h):
    D, B = h.shape
    BM, BN = 512, 256
    return pl.pallas_call(
        _transpose_body,
        grid=(D // BM, B // BN),
        in_specs=[pl.BlockSpec((BM, BN), lambda i, j: (i, j))],
        out_specs=pl.BlockSpec((BN, BM), lambda i, j: (j, i)),
        out_shape=jax.ShapeDtypeStruct((B, D), jnp.float32),
    )(h)


def kernel(x, W0, b0, vals1, rows1, cols1, vals2, rows2, cols2):
    S1 = 16384
    S2 = 4096
    h1 = _dense_tc(x, W0, b0)
    h2 = _spmm_sc(h1, vals1, rows1, cols1, M=S1, RB=32, CH=32, relu_in=False)
    h3 = _spmm_sc(h2, vals2, rows2, cols2, M=S2, RB=32, CH=32, relu_in=True)
    return _transpose_tc(h3)


# speedup vs baseline: 3.9826x; 1.1493x over previous
import functools

import jax
import jax.numpy as jnp
from jax import lax
from jax.experimental import pallas as pl
from jax.experimental.pallas import tpu as pltpu
from jax.experimental.pallas import tpu_sc as plsc

BATCH = 1024
NC = 2
NS = 16
NW = NC * NS


def _dense_body(w_ref, x_ref, b_ref, out_ref):
    i = pl.program_id(0)
    acc = lax.dot_general(
        w_ref[...], x_ref[...],
        dimension_numbers=(((1,), (1,)), ((), ())),
        preferred_element_type=jnp.float32,
    )
    bias = b_ref[0, pl.ds(i * w_ref.shape[0], w_ref.shape[0])]
    out_ref[...] = jnp.maximum(acc + bias[:, None], 0.0)


def _dense_tc(x, W0, b0):
    D, K = W0.shape
    BM = 256
    grid = (D // BM,)
    return pl.pallas_call(
        _dense_body,
        grid=grid,
        in_specs=[
            pl.BlockSpec((BM, K), lambda i: (i, 0)),
            pl.BlockSpec((BATCH, K), lambda i: (0, 0)),
            pl.BlockSpec((1, D), lambda i: (0, 0)),
        ],
        out_specs=pl.BlockSpec((BM, BATCH), lambda i: (i, 0)),
        out_shape=jax.ShapeDtypeStruct((D, BATCH), jnp.float32),
    )(W0, x, b0.reshape(1, D))


def _spmm_body(M, RB, NB, CH, RPW, relu_in,
               h_hbm, rv_hbm, cols_hbm, bptr_hbm, out_hbm,
               bptr_v, cbb0, cbb1, rvb0, rvb1, gb0, gb1, acc,
               sg0, sg1, scb0, scb1, srv0, srv1):
    wid = lax.axis_index("s") * NC + lax.axis_index("c")
    pltpu.sync_copy(bptr_hbm, bptr_v)
    zvec = jnp.zeros((16,), jnp.float32)

    def blk_body(blk, _):
        r0 = wid * RPW + blk * RB
        bp = bptr_v[pl.ds(wid * NB + blk, 16)]
        e0 = bp[0]
        e1 = bp[1]
        e0a = (e0 // 8) * 8
        nch = jnp.maximum((e1 - e0a + CH - 1) // CH, 1)
        nchp = (nch + 1) // 2

        def st(c):
            return e0a + jnp.minimum(c, nch - 1) * CH

        def cb_copy(c, cbb, sem):
            return pltpu.make_async_copy(
                cols_hbm.at[pl.ds(st(c), CH)], cbb, sem)

        def rv_copy(c, rvb, sem):
            return pltpu.make_async_copy(
                rv_hbm.at[pl.ds(2 * st(c), 2 * CH)],
                rvb.at[pl.ds(0, 2 * CH)], sem)

        def g_copy(cbb, gb, sem):
            return pltpu.make_async_copy(h_hbm.at[cbb], gb, sem)

        cb_copy(0, cbb0, scb0).start()
        rv_copy(0, rvb0, srv0).start()
        cb_copy(1, cbb1, scb1).start()
        rv_copy(1, rvb1, srv1).start()

        @plsc.parallel_loop(0, RB * BATCH // 16, 1, unroll=8)
        def _zero(i):
            acc[pl.ds(i * 16, 16)] = zvec
        cb_copy(0, cbb0, scb0).wait()
        g_copy(cbb0, gb0, sg0).start()

        def compute(gb, rvb, guard):
            def ent(e, _):
                w = rvb[pl.ds(2 * e, 16)]
                lr = w[0] - r0
                ok = jnp.logical_and(
                    jnp.logical_and(lr >= 0, lr < RB), guard)
                v = jnp.where(ok, lax.bitcast_convert_type(w[1], jnp.float32),
                              0.0)
                lrc = jnp.where(ok, lr, 0)
                vvec = jnp.full((16,), v, jnp.float32)
                base = lrc * BATCH

                @plsc.parallel_loop(0, BATCH // 16, 1, unroll=8)
                def _col(j):
                    g = gb[e, pl.ds(j * 16, 16)]
                    if relu_in:
                        g = jnp.maximum(g, 0.0)
                    plsc.addupdate(acc.at[pl.ds(base + j * 16, 16)], vvec * g)

                return 0

            lax.fori_loop(0, CH, ent, 0)

        def pair(t, _):
            a = 2 * t
            b = a + 1
            g_copy(cbb0, gb0, sg0).wait()
            cb_copy(b, cbb1, scb1).wait()
            g_copy(cbb1, gb1, sg1).start()
            cb_copy(a + 2, cbb0, scb0).start()
            rv_copy(a, rvb0, srv0).wait()
            compute(gb0, rvb0, True)
            rv_copy(a + 2, rvb0, srv0).start()
            g_copy(cbb1, gb1, sg1).wait()
            cb_copy(a + 2, cbb0, scb0).wait()
            g_copy(cbb0, gb0, sg0).start()
            cb_copy(b + 2, cbb1, scb1).start()
            rv_copy(b, rvb1, srv1).wait()
            compute(gb1, rvb1, b < nch)
            rv_copy(b + 2, rvb1, srv1).start()
            return 0

        lax.fori_loop(0, nchp, pair, 0)
        g_copy(cbb0, gb0, sg0).wait()
        cb_copy(0, cbb1, scb1).wait()
        rv_copy(0, rvb0, srv0).wait()
        rv_copy(0, rvb1, srv1).wait()
        pltpu.sync_copy(acc, out_hbm.at[pl.ds(r0 * BATCH, RB * BATCH)])
        return 0

    lax.fori_loop(0, NB, blk_body, 0)


def _spmm_sc(h, vals, rows, cols, M, RB, CH, relu_in):
    nnz = vals.shape[0]
    RPW = M // NW
    NB = RPW // RB
    bptr = jnp.searchsorted(rows, jnp.arange(0, M + 1, RB)).astype(jnp.int32)
    PB = ((M // RB + 1 + 16 + 7) // 8) * 8
    bptr = jnp.concatenate(
        [bptr, jnp.full((PB - bptr.shape[0],), nnz, jnp.int32)])
    L = (((nnz + 2 * CH) + 7) // 8) * 8
    pad = L - nnz
    rows_p = jnp.concatenate([rows, jnp.full((pad,), M, jnp.int32)])
    cols_p = jnp.concatenate([cols, jnp.zeros((pad,), jnp.int32)])
    vals_p = jnp.concatenate([vals, jnp.zeros((pad,), jnp.float32)])
    vbits = lax.bitcast_convert_type(vals_p, jnp.int32)
    rv = jnp.stack([rows_p, vbits], axis=1).reshape(-1)

    mesh = plsc.VectorSubcoreMesh(core_axis_name="c", subcore_axis_name="s",
                                  num_cores=NC, num_subcores=NS)
    body = functools.partial(_spmm_body, M, RB, NB, CH, RPW, relu_in)
    f = pl.kernel(
        body,
        out_type=jax.ShapeDtypeStruct((M * BATCH,), jnp.float32),
        mesh=mesh,
        scratch_types=[
            pltpu.VMEM((PB,), jnp.int32),
            pltpu.VMEM((CH,), jnp.int32),
            pltpu.VMEM((CH,), jnp.int32),
            pltpu.VMEM((2 * CH + 16,), jnp.int32),
            pltpu.VMEM((2 * CH + 16,), jnp.int32),
            pltpu.VMEM((CH, BATCH), jnp.float32),
            pltpu.VMEM((CH, BATCH), jnp.float32),
            pltpu.VMEM((RB * BATCH,), jnp.float32),
            pltpu.SemaphoreType.DMA,
            pltpu.SemaphoreType.DMA,
            pltpu.SemaphoreType.DMA,
            pltpu.SemaphoreType.DMA,
            pltpu.SemaphoreType.DMA,
            pltpu.SemaphoreType.DMA,
        ],
    )
    out = f(h, rv, cols_p, bptr)
    return out.reshape(M, BATCH)


def _transpose_body(in_ref, out_ref):
    out_ref[...] = jnp.maximum(in_ref[...].T, 0.0)


def _transpose_tc(h):
    D, B = h.shape
    BM, BN = 512, 256
    return pl.pallas_call(
        _transpose_body,
        grid=(D // BM, B // BN),
        in_specs=[pl.BlockSpec((BM, BN), lambda i, j: (i, j))],
        out_specs=pl.BlockSpec((BN, BM), lambda i, j: (j, i)),
        out_shape=jax.ShapeDtypeStruct((B, D), jnp.float32),
    )(h)


def kernel(x, W0, b0, vals1, rows1, cols1, vals2, rows2, cols2):
    S1 = 16384
    S2 = 4096
    h1 = _dense_tc(x, W0, b0)
    h2 = _spmm_sc(h1, vals1, rows1, cols1, M=S1, RB=64, CH=24, relu_in=False)
    h3 = _spmm_sc(h2, vals2, rows2, cols2, M=S2, RB=64, CH=24, relu_in=True)
    return _transpose_tc(h3)


# speedup vs baseline: 4.3127x; 1.0829x over previous
import functools

import jax
import jax.numpy as jnp
from jax import lax
from jax.experimental import pallas as pl
from jax.experimental.pallas import tpu as pltpu
from jax.experimental.pallas import tpu_sc as plsc

BATCH = 1024
NC = 2
NS = 16
NW = NC * NS


def _dense_body(w_ref, x_ref, b_ref, out_ref):
    i = pl.program_id(0)
    acc = lax.dot_general(
        w_ref[...], x_ref[...],
        dimension_numbers=(((1,), (1,)), ((), ())),
        preferred_element_type=jnp.float32,
    )
    bias = b_ref[0, pl.ds(i * w_ref.shape[0], w_ref.shape[0])]
    out_ref[...] = jnp.maximum(acc + bias[:, None], 0.0)


def _dense_tc(x, W0, b0):
    D, K = W0.shape
    BM = 256
    grid = (D // BM,)
    return pl.pallas_call(
        _dense_body,
        grid=grid,
        in_specs=[
            pl.BlockSpec((BM, K), lambda i: (i, 0)),
            pl.BlockSpec((BATCH, K), lambda i: (0, 0)),
            pl.BlockSpec((1, D), lambda i: (0, 0)),
        ],
        out_specs=pl.BlockSpec((BM, BATCH), lambda i: (i, 0)),
        out_shape=jax.ShapeDtypeStruct((D, BATCH), jnp.float32),
    )(W0, x, b0.reshape(1, D))


def _spmm_body(M, RB, NB, CH, RPW, relu_in,
               h_hbm, rv_hbm, cols_hbm, bptr_hbm, out_hbm,
               bptr_v, cbb0, cbb1, rvb0, rvb1, gb0, gb1, acc,
               sg0, sg1, scb0, scb1, srv0, srv1):
    wid = lax.axis_index("s") * NC + lax.axis_index("c")
    pltpu.sync_copy(bptr_hbm, bptr_v)
    zvec = jnp.zeros((16,), jnp.float32)
    JW = BATCH // 16

    def blk_body(blk, _):
        r0 = wid * RPW + blk * RB
        bp = bptr_v[pl.ds(wid * NB + blk, 16)]
        e0 = bp[0]
        e1 = bp[1]
        e0a = (e0 // 8) * 8
        nch = jnp.maximum((e1 - e0a + CH - 1) // CH, 1)
        nchp = (nch + 1) // 2

        def st(c):
            return e0a + jnp.minimum(c, nch - 1) * CH

        def cb_copy(c, cbb, sem):
            return pltpu.make_async_copy(
                cols_hbm.at[pl.ds(st(c), CH)], cbb, sem)

        def rv_copy(c, rvb, sem):
            return pltpu.make_async_copy(
                rv_hbm.at[pl.ds(2 * st(c), 2 * CH)],
                rvb.at[pl.ds(0, 2 * CH)], sem)

        def g_copy(cbb, gb, sem):
            return pltpu.make_async_copy(h_hbm.at[cbb], gb, sem)

        cb_copy(0, cbb0, scb0).start()
        rv_copy(0, rvb0, srv0).start()
        cb_copy(1, cbb1, scb1).start()
        rv_copy(1, rvb1, srv1).start()

        @plsc.parallel_loop(0, RB * JW, 1, unroll=8)
        def _zero(i):
            acc[i // JW, pl.ds((i % JW) * 16, 16)] = zvec
        cb_copy(0, cbb0, scb0).wait()
        g_copy(cbb0, gb0, sg0).start()

        def compute(gb, rvb, guard):
            def ent(e, _):
                w = rvb[pl.ds(2 * e, 16)]
                lr = w[0] - r0
                ok = jnp.logical_and(
                    jnp.logical_and(lr >= 0, lr < RB), guard)
                v = jnp.where(ok, lax.bitcast_convert_type(w[1], jnp.float32),
                              0.0)
                lrc = jnp.where(ok, lr, 0)
                vvec = jnp.full((16,), v, jnp.float32)

                @plsc.parallel_loop(0, JW, 1, unroll=8)
                def _col(j):
                    g = gb[e, pl.ds(j * 16, 16)]
                    if relu_in:
                        g = jnp.maximum(g, 0.0)
                    plsc.addupdate(acc.at[lrc, pl.ds(j * 16, 16)], vvec * g)

                return 0

            lax.fori_loop(0, CH, ent, 0)

        def pair(t, _):
            a = 2 * t
            b = a + 1
            g_copy(cbb0, gb0, sg0).wait()
            cb_copy(b, cbb1, scb1).wait()
            g_copy(cbb1, gb1, sg1).start()
            cb_copy(a + 2, cbb0, scb0).start()
            rv_copy(a, rvb0, srv0).wait()
            compute(gb0, rvb0, True)
            rv_copy(a + 2, rvb0, srv0).start()
            g_copy(cbb1, gb1, sg1).wait()
            cb_copy(a + 2, cbb0, scb0).wait()
            g_copy(cbb0, gb0, sg0).start()
            cb_copy(b + 2, cbb1, scb1).start()
            rv_copy(b, rvb1, srv1).wait()
            compute(gb1, rvb1, b < nch)
            rv_copy(b + 2, rvb1, srv1).start()
            return 0

        lax.fori_loop(0, nchp, pair, 0)
        g_copy(cbb0, gb0, sg0).wait()
        cb_copy(0, cbb1, scb1).wait()
        rv_copy(0, rvb0, srv0).wait()
        rv_copy(0, rvb1, srv1).wait()
        pltpu.sync_copy(acc, out_hbm.at[pl.ds(r0, RB)])
        return 0

    lax.fori_loop(0, NB, blk_body, 0)


def _spmm_sc(h, vals, rows, cols, M, RB, CH, relu_in):
    nnz = vals.shape[0]
    RPW = M // NW
    NB = RPW // RB
    bptr = jnp.searchsorted(rows, jnp.arange(0, M + 1, RB)).astype(jnp.int32)
    PB = ((M // RB + 1 + 16 + 7) // 8) * 8
    bptr = jnp.concatenate(
        [bptr, jnp.full((PB - bptr.shape[0],), nnz, jnp.int32)])
    L = (((nnz + 2 * CH) + 7) // 8) * 8
    pad = L - nnz
    rows_p = jnp.concatenate([rows, jnp.full((pad,), M, jnp.int32)])
    cols_p = jnp.concatenate([cols, jnp.zeros((pad,), jnp.int32)])
    vals_p = jnp.concatenate([vals, jnp.zeros((pad,), jnp.float32)])
    vbits = lax.bitcast_convert_type(vals_p, jnp.int32)
    rv = jnp.stack([rows_p, vbits], axis=1).reshape(-1)

    mesh = plsc.VectorSubcoreMesh(core_axis_name="c", subcore_axis_name="s",
                                  num_cores=NC, num_subcores=NS)
    body = functools.partial(_spmm_body, M, RB, NB, CH, RPW, relu_in)
    f = pl.kernel(
        body,
        out_type=jax.ShapeDtypeStruct((M, BATCH), jnp.float32),
        mesh=mesh,
        scratch_types=[
            pltpu.VMEM((PB,), jnp.int32),
            pltpu.VMEM((CH,), jnp.int32),
            pltpu.VMEM((CH,), jnp.int32),
            pltpu.VMEM((2 * CH + 16,), jnp.int32),
            pltpu.VMEM((2 * CH + 16,), jnp.int32),
            pltpu.VMEM((CH, BATCH), jnp.float32),
            pltpu.VMEM((CH, BATCH), jnp.float32),
            pltpu.VMEM((RB, BATCH), jnp.float32),
            pltpu.SemaphoreType.DMA,
            pltpu.SemaphoreType.DMA,
            pltpu.SemaphoreType.DMA,
            pltpu.SemaphoreType.DMA,
            pltpu.SemaphoreType.DMA,
            pltpu.SemaphoreType.DMA,
        ],
    )
    return f(h, rv, cols_p, bptr)


def _transpose_body(in_ref, out_ref):
    out_ref[...] = jnp.maximum(in_ref[...].T, 0.0)


def _transpose_tc(h):
    D, B = h.shape
    BM, BN = 512, 256
    return pl.pallas_call(
        _transpose_body,
        grid=(D // BM, B // BN),
        in_specs=[pl.BlockSpec((BM, BN), lambda i, j: (i, j))],
        out_specs=pl.BlockSpec((BN, BM), lambda i, j: (j, i)),
        out_shape=jax.ShapeDtypeStruct((B, D), jnp.float32),
    )(h)


def kernel(x, W0, b0, vals1, rows1, cols1, vals2, rows2, cols2):
    S1 = 16384
    S2 = 4096
    h1 = _dense_tc(x, W0, b0)
    h2 = _spmm_sc(h1, vals1, rows1, cols1, M=S1, RB=64, CH=24, relu_in=False)
    h3 = _spmm_sc(h2, vals2, rows2, cols2, M=S2, RB=64, CH=24, relu_in=True)
    return _transpose_tc(h3)


# speedup vs baseline: 4.3139x; 1.0003x over previous
import functools

import jax
import jax.numpy as jnp
from jax import lax
from jax.experimental import pallas as pl
from jax.experimental.pallas import tpu as pltpu
from jax.experimental.pallas import tpu_sc as plsc

BATCH = 1024
NC = 2
NS = 16
NW = NC * NS


def _dense_body(w_ref, x_ref, b_ref, out_ref):
    i = pl.program_id(0)
    acc = lax.dot_general(
        w_ref[...].astype(jnp.bfloat16), x_ref[...].astype(jnp.bfloat16),
        dimension_numbers=(((1,), (1,)), ((), ())),
        preferred_element_type=jnp.float32,
    )
    bias = b_ref[0, pl.ds(i * w_ref.shape[0], w_ref.shape[0])]
    out_ref[...] = jnp.maximum(acc + bias[:, None], 0.0)


def _dense_tc(x, W0, b0):
    D, K = W0.shape
    BM = 256
    grid = (D // BM,)
    return pl.pallas_call(
        _dense_body,
        grid=grid,
        in_specs=[
            pl.BlockSpec((BM, K), lambda i: (i, 0)),
            pl.BlockSpec((BATCH, K), lambda i: (0, 0)),
            pl.BlockSpec((1, D), lambda i: (0, 0)),
        ],
        out_specs=pl.BlockSpec((BM, BATCH), lambda i: (i, 0)),
        out_shape=jax.ShapeDtypeStruct((D, BATCH), jnp.float32),
    )(W0, x, b0.reshape(1, D))


def _spmm_body(M, RB, NB, CH, RPW, relu_in,
               h_hbm, rv_hbm, cols_hbm, bptr_hbm, out_hbm,
               bptr_v, cbb0, cbb1, rvb0, rvb1, gb0, gb1, acc,
               sg0, sg1, scb0, scb1, srv0, srv1):
    wid = lax.axis_index("s") * NC + lax.axis_index("c")
    pltpu.sync_copy(bptr_hbm, bptr_v)
    zvec = jnp.zeros((16,), jnp.float32)
    JW = BATCH // 16

    def blk_body(blk, _):
        r0 = wid * RPW + blk * RB
        bp = bptr_v[pl.ds(wid * NB + blk, 16)]
        e0 = bp[0]
        e1 = bp[1]
        e0a = (e0 // 8) * 8
        nch = jnp.maximum((e1 - e0a + CH - 1) // CH, 1)
        nchp = (nch + 1) // 2

        def st(c):
            return e0a + jnp.minimum(c, nch - 1) * CH

        def cb_copy(c, cbb, sem):
            return pltpu.make_async_copy(
                cols_hbm.at[pl.ds(st(c), CH)], cbb, sem)

        def rv_copy(c, rvb, sem):
            return pltpu.make_async_copy(
                rv_hbm.at[pl.ds(2 * st(c), 2 * CH)],
                rvb.at[pl.ds(0, 2 * CH)], sem)

        def g_copy(cbb, gb, sem):
            return pltpu.make_async_copy(h_hbm.at[cbb], gb, sem)

        cb_copy(0, cbb0, scb0).start()
        rv_copy(0, rvb0, srv0).start()
        cb_copy(1, cbb1, scb1).start()
        rv_copy(1, rvb1, srv1).start()

        @plsc.parallel_loop(0, RB * JW, 1, unroll=8)
        def _zero(i):
            acc[i // JW, pl.ds((i % JW) * 16, 16)] = zvec
        cb_copy(0, cbb0, scb0).wait()
        g_copy(cbb0, gb0, sg0).start()

        def compute(gb, rvb, guard):
            def ent(e, _):
                w = rvb[pl.ds(2 * e, 16)]
                lr = w[0] - r0
                ok = jnp.logical_and(
                    jnp.logical_and(lr >= 0, lr < RB), guard)
                v = jnp.where(ok, lax.bitcast_convert_type(w[1], jnp.float32),
                              0.0)
                lrc = jnp.where(ok, lr, 0)
                vvec = jnp.full((16,), v, jnp.float32)

                @plsc.parallel_loop(0, JW, 1, unroll=8)
                def _col(j):
                    g = gb[e, pl.ds(j * 16, 16)]
                    if relu_in:
                        g = jnp.maximum(g, 0.0)
                    plsc.addupdate(acc.at[lrc, pl.ds(j * 16, 16)], vvec * g)

                return 0

            lax.fori_loop(0, CH, ent, 0)

        def pair(t, _):
            a = 2 * t
            b = a + 1
            g_copy(cbb0, gb0, sg0).wait()
            cb_copy(b, cbb1, scb1).wait()
            g_copy(cbb1, gb1, sg1).start()
            cb_copy(a + 2, cbb0, scb0).start()
            rv_copy(a, rvb0, srv0).wait()
            compute(gb0, rvb0, True)
            rv_copy(a + 2, rvb0, srv0).start()
            g_copy(cbb1, gb1, sg1).wait()
            cb_copy(a + 2, cbb0, scb0).wait()
            g_copy(cbb0, gb0, sg0).start()
            cb_copy(b + 2, cbb1, scb1).start()
            rv_copy(b, rvb1, srv1).wait()
            compute(gb1, rvb1, b < nch)
            rv_copy(b + 2, rvb1, srv1).start()
            return 0

        lax.fori_loop(0, nchp, pair, 0)
        g_copy(cbb0, gb0, sg0).wait()
        cb_copy(0, cbb1, scb1).wait()
        rv_copy(0, rvb0, srv0).wait()
        rv_copy(0, rvb1, srv1).wait()
        pltpu.sync_copy(acc, out_hbm.at[pl.ds(r0, RB)])
        return 0

    lax.fori_loop(0, NB, blk_body, 0)


def _spmm_sc(h, vals, rows, cols, M, RB, CH, relu_in):
    nnz = vals.shape[0]
    RPW = M // NW
    NB = RPW // RB
    bptr = jnp.searchsorted(rows, jnp.arange(0, M + 1, RB)).astype(jnp.int32)
    PB = ((M // RB + 1 + 16 + 7) // 8) * 8
    bptr = jnp.concatenate(
        [bptr, jnp.full((PB - bptr.shape[0],), nnz, jnp.int32)])
    L = (((nnz + 2 * CH) + 7) // 8) * 8
    pad = L - nnz
    rows_p = jnp.concatenate([rows, jnp.full((pad,), M, jnp.int32)])
    cols_p = jnp.concatenate([cols, jnp.zeros((pad,), jnp.int32)])
    vals_p = jnp.concatenate([vals, jnp.zeros((pad,), jnp.float32)])
    vbits = lax.bitcast_convert_type(vals_p, jnp.int32)
    rv = jnp.stack([rows_p, vbits], axis=1).reshape(-1)

    mesh = plsc.VectorSubcoreMesh(core_axis_name="c", subcore_axis_name="s",
                                  num_cores=NC, num_subcores=NS)
    body = functools.partial(_spmm_body, M, RB, NB, CH, RPW, relu_in)
    f = pl.kernel(
        body,
        out_type=jax.ShapeDtypeStruct((M, BATCH), jnp.float32),
        mesh=mesh,
        scratch_types=[
            pltpu.VMEM((PB,), jnp.int32),
            pltpu.VMEM((CH,), jnp.int32),
            pltpu.VMEM((CH,), jnp.int32),
            pltpu.VMEM((2 * CH + 16,), jnp.int32),
            pltpu.VMEM((2 * CH + 16,), jnp.int32),
            pltpu.VMEM((CH, BATCH), jnp.float32),
            pltpu.VMEM((CH, BATCH), jnp.float32),
            pltpu.VMEM((RB, BATCH), jnp.float32),
            pltpu.SemaphoreType.DMA,
            pltpu.SemaphoreType.DMA,
            pltpu.SemaphoreType.DMA,
            pltpu.SemaphoreType.DMA,
            pltpu.SemaphoreType.DMA,
            pltpu.SemaphoreType.DMA,
        ],
    )
    return f(h, rv, cols_p, bptr)


def _transpose_body(in_ref, out_ref):
    out_ref[...] = jnp.maximum(in_ref[...].T, 0.0)


def _transpose_tc(h):
    D, B = h.shape
    BM, BN = 512, 256
    return pl.pallas_call(
        _transpose_body,
        grid=(D // BM, B // BN),
        in_specs=[pl.BlockSpec((BM, BN), lambda i, j: (i, j))],
        out_specs=pl.BlockSpec((BN, BM), lambda i, j: (j, i)),
        out_shape=jax.ShapeDtypeStruct((B, D), jnp.float32),
    )(h)


def kernel(x, W0, b0, vals1, rows1, cols1, vals2, rows2, cols2):
    S1 = 16384
    S2 = 4096
    h1 = _dense_tc(x, W0, b0)
    h2 = _spmm_sc(h1, vals1, rows1, cols1, M=S1, RB=64, CH=24, relu_in=False)
    h3 = _spmm_sc(h2, vals2, rows2, cols2, M=S2, RB=64, CH=24, relu_in=True)
    return _transpose_tc(h3)
